# baseline (speedup 1.0000x reference)
import dataclasses
import functools

import jax
import jax.numpy as jnp
from jax import lax
from jax.experimental import pallas as pl
from jax.experimental.pallas import tpu as pltpu
from jax.experimental.pallas import tpu_sc as plsc

_NC = 2
_NS = 16
_NW = _NC * _NS
_LANE = 128


def _round_up(v, m):
    return (v + m - 1) // m * m


def _layer_tail(n, h, wp_ref, as_ref, ad_ref, h_ref, hpb_ref, asp_ref,
                adp_ref, m_ref):
    h_ref[...] = h
    hpp = jnp.dot(h, wp_ref[...], preferred_element_type=jnp.float32)
    a_s = jnp.sum(hpp * as_ref[...], axis=1, keepdims=True)
    a_d = jnp.sum(hpp * ad_ref[...], axis=1, keepdims=True)
    asp_ref[0:n, :] = a_s
    asp_ref[n:, :] = jnp.full((asp_ref.shape[0] - n, 1), -1e30, jnp.float32)
    adp_ref[...] = a_d
    hpb_ref[0:n, :] = hpp.astype(jnp.bfloat16)
    hpb_ref[n:, :] = jnp.zeros((hpb_ref.shape[0] - n, hpp.shape[1]),
                               jnp.bfloat16)
    mm = jnp.max(a_s) + jnp.max(a_d)
    m_ref[...] = jnp.broadcast_to(jnp.where(mm >= 0, mm, 0.2 * mm), (1, 16))


def _first_body(n, x_ref, w0_ref, b0_ref, wp_ref, as_ref, ad_ref,
                h_ref, hpb_ref, asp_ref, adp_ref, m_ref):
    acc = jnp.dot(x_ref[...], w0_ref[...],
                  preferred_element_type=jnp.float32)
    h = jnp.maximum(acc + b0_ref[...], 0.0)
    _layer_tail(n, h, wp_ref, as_ref, ad_ref, h_ref, hpb_ref, asp_ref,
                adp_ref, m_ref)


def _mid_body(n, h0_ref, msg_ref, den_ref, b_ref, wp_ref, as_ref, ad_ref,
              h_ref, hpb_ref, asp_ref, adp_ref, m_ref):
    sm = msg_ref[0] + msg_ref[1]
    d = den_ref[0] + den_ref[1]
    h = h0_ref[...] + jnp.maximum(sm / (d + 1e-16) + b_ref[...], 0.0)
    _layer_tail(n, h, wp_ref, as_ref, ad_ref, h_ref, hpb_ref, asp_ref,
                adp_ref, m_ref)


def _epilogue_body(h_ref, msg_ref, den_ref, b_ref, o_ref):
    sm = msg_ref[0] + msg_ref[1]
    d = den_ref[0] + den_ref[1]
    o_ref[...] = h_ref[...] + jnp.maximum(sm / (d + 1e-16) + b_ref[...], 0.0)


def _make_sc_edge_pass(n, f, rw):
    zc = (n // _NS) & ~7
    zt = n - _NS * zc
    on = n // _NS
    mesh = plsc.VectorSubcoreMesh(core_axis_name="c", subcore_axis_name="s")
    cp = pltpu.CompilerParams()
    if "needs_layout_passes" in pltpu.CompilerParams.__dataclass_fields__:
        cp = dataclasses.replace(cp, needs_layout_passes=False)
    if "use_tc_tiling_on_sc" in pltpu.CompilerParams.__dataclass_fields__:
        cp = dataclasses.replace(cp, use_tc_tiling_on_sc=False)

    @functools.partial(
        pl.kernel,
        mesh=mesh,
        compiler_params=cp,
        out_type=[
            jax.ShapeDtypeStruct((_NC, n, f), jnp.float32),
            jax.ShapeDtypeStruct((_NC, n), jnp.float32),
        ],
        scratch_types=[
            pltpu.VMEM((n + 16,), jnp.float32),
            pltpu.VMEM((n,), jnp.float32),
            pltpu.VMEM((16,), jnp.float32),
            pltpu.VMEM((rw, _LANE), jnp.int32),
            pltpu.VMEM((rw, _LANE), jnp.int32),
            pltpu.VMEM((4, _LANE, f), jnp.bfloat16),
            pltpu.VMEM((4, _LANE, f), jnp.float32),
            pltpu.VMEM((4, _LANE), jnp.float32),
            pltpu.VMEM_SHARED((n, f), jnp.float32),
            pltpu.VMEM_SHARED((n,), jnp.float32),
            pltpu.SemaphoreType.DMA((4,)),
            pltpu.SemaphoreType.DMA((4,)),
            pltpu.SemaphoreType.DMA((4,)),
        ],
    )
    def sc_edge_pass(hp_hbm, asrc_hbm, adst_hbm, m_hbm, src_hbm, dst_hbm,
                     z2_hbm, z1_hbm, msg_out, den_out, asrc_v, adst_v, m_v,
                     src_v, dst_v, rows_v, scat_v, ex_v, msg_acc, den_acc,
                     gsem, msem, dsem):
        c = lax.axis_index("c")
        s = lax.axis_index("s")
        w = s * _NC + c
        stage = [
            pltpu.async_copy(z2_hbm, msg_acc.at[pl.ds(s * zc, zc)],
                             msem.at[0]),
            pltpu.async_copy(z1_hbm, den_acc.at[pl.ds(s * zc, zc)],
                             msem.at[1]),
            pltpu.async_copy(asrc_hbm, asrc_v, msem.at[2]),
            pltpu.async_copy(adst_hbm, adst_v, msem.at[3]),
            pltpu.async_copy(m_hbm, m_v, dsem.at[0]),
            pltpu.async_copy(src_hbm.at[pl.ds(w * rw, rw)], src_v,
                             dsem.at[1]),
            pltpu.async_copy(dst_hbm.at[pl.ds(w * rw, rw)], dst_v,
                             dsem.at[2]),
        ]
        if zt:
            @pl.when(s == 0)
            def _zero_tail():
                pltpu.async_copy(z2_hbm.at[pl.ds(0, zt)],
                                 msg_acc.at[pl.ds(_NS * zc, zt)],
                                 dsem.at[3]).wait()
                pltpu.async_copy(z1_hbm.at[pl.ds(0, zt)],
                                 den_acc.at[pl.ds(_NS * zc, zt)],
                                 gsem.at[0]).wait()
        for cp in stage:
            cp.wait()
        plsc.subcore_barrier()
        mvec = m_v[...]

        def _drain_scatters(b, r):
            pltpu.make_async_copy(scat_v.at[b], msg_acc.at[dst_v.at[r]],
                                  msem.at[b]).wait()
            pltpu.make_async_copy(ex_v.at[b], den_acc.at[dst_v.at[r]],
                                  dsem.at[b]).wait()

        def _do_row(r, b):
            nb = (b + 3) % 4

            @pl.when(r + 3 < rw)
            def _prefetch():
                pltpu.async_copy(hp_hbm.at[src_v.at[r + 3].at[pl.ds(0, 64)]],
                                 rows_v.at[nb].at[pl.ds(0, 64)], gsem.at[nb])
                pltpu.async_copy(hp_hbm.at[src_v.at[r + 3].at[pl.ds(64, 64)]],
                                 rows_v.at[nb].at[pl.ds(64, 64)], gsem.at[nb])

            pltpu.make_async_copy(hp_hbm.at[src_v.at[r]], rows_v.at[b],
                                  gsem.at[b]).wait()

            @pl.when(r >= 4)
            def _drain():
                _drain_scatters(b, r)

            exb = ex_v.at[b]
            for k in range(_LANE // 16):
                sidx = src_v[r, pl.ds(k * 16, 16)]
                didx = dst_v[r, pl.ds(k * 16, 16)]
                a = (plsc.load_gather(asrc_v, [sidx])
                     + plsc.load_gather(adst_v, [didx]))
                a = jnp.where(a >= 0, a, a * 0.2)
                exb[pl.ds(k * 16, 16)] = jnp.exp(a - mvec)

            rv = rows_v.at[b]
            sv = scat_v.at[b]
            himask = jnp.full((16,), 0xFFFF0000, jnp.uint32)
            for i in range(_LANE):
                eb = plsc.load_gather(exb, [jnp.full((16,), i, jnp.int32)])
                for h in range(f // 32):
                    u = plsc.bitcast(rv[i, pl.ds(h * 32, 32)], jnp.uint32)
                    lo = plsc.bitcast(u << 16, jnp.float32)
                    hi = plsc.bitcast(u & himask, jnp.float32)
                    sv[i, pl.ds(h * 32, 16)] = lo * eb
                    sv[i, pl.ds(h * 32 + 16, 16)] = hi * eb

            pltpu.async_copy(exb, den_acc.at[dst_v.at[r]], dsem.at[b],
                             add=True)
            pltpu.async_copy(sv, msg_acc.at[dst_v.at[r]], msem.at[b],
                             add=True)

        for pb in range(3):
            pltpu.async_copy(hp_hbm.at[src_v.at[pb].at[pl.ds(0, 64)]],
                             rows_v.at[pb].at[pl.ds(0, 64)], gsem.at[pb])
            pltpu.async_copy(hp_hbm.at[src_v.at[pb].at[pl.ds(64, 64)]],
                             rows_v.at[pb].at[pl.ds(64, 64)], gsem.at[pb])

        @pl.loop(0, rw // 4)
        def _edges(q):
            _do_row(4 * q, 0)
            _do_row(4 * q + 1, 1)
            _do_row(4 * q + 2, 2)
            _do_row(4 * q + 3, 3)

        for b in range(4):
            _drain_scatters(b, 0)
        plsc.subcore_barrier()
        sl = pl.ds(s * on, on)
        pltpu.sync_copy(msg_acc.at[sl], msg_out.at[c].at[sl])

        @pl.when(s == 0)
        def _den_out():
            pltpu.sync_copy(den_acc.at[pl.ds(0, n)], den_out.at[c])

    return sc_edge_pass


def kernel(x, edge_index, edge_weight, edge_attr, W0, b0, W1, att_src1,
           att_dst1, b1, W2, att_src2, att_dst2, b2):
    n, d = x.shape
    f = W0.shape[1]
    e = edge_index.shape[1]

    rows = _round_up(pl.cdiv(e, _LANE), _NW * 8)
    rw = rows // _NW
    ep = rows * _LANE

    src = edge_index[0]
    dst = edge_index[1]
    pad = ep - e
    src2d = jnp.concatenate([src, jnp.full((pad,), n, jnp.int32)]).reshape(
        rows, _LANE)
    dst2d = jnp.concatenate([dst, jnp.zeros((pad,), jnp.int32)]).reshape(
        rows, _LANE)
    z2 = jnp.zeros(((n // _NS) & ~7, f), jnp.float32)
    z1 = jnp.zeros(((n // _NS) & ~7,), jnp.float32)
    perm = [0] * f
    for h in range(f // 32):
        for k in range(16):
            perm[32 * h + 2 * k] = 32 * h + k
            perm[32 * h + 2 * k + 1] = 32 * h + 16 + k
    perm = jnp.asarray(perm, jnp.int32)

    sc_edge_pass = _make_sc_edge_pass(n, f, rw)

    pre_out = [
        jax.ShapeDtypeStruct((n, f), jnp.float32),
        jax.ShapeDtypeStruct((n + 16, f), jnp.bfloat16),
        jax.ShapeDtypeStruct((n + 16, 1), jnp.float32),
        jax.ShapeDtypeStruct((n, 1), jnp.float32),
        jax.ShapeDtypeStruct((1, 16), jnp.float32),
    ]
    first = pl.pallas_call(functools.partial(_first_body, n),
                           out_shape=pre_out)
    mid = pl.pallas_call(functools.partial(_mid_body, n), out_shape=pre_out)
    epilogue = pl.pallas_call(
        _epilogue_body,
        out_shape=jax.ShapeDtypeStruct((n, f), jnp.float32),
    )

    def run_sc(hpb, a_s, a_d, m):
        return sc_edge_pass(hpb, a_s.reshape(n + 16), a_d.reshape(n),
                            m.reshape(16), src2d, dst2d, z2, z1)

    h0, hpb1, as1, ad1, m1 = first(x, W0, b0.reshape(1, f), W1[:, perm],
                                   att_src1[perm].reshape(1, f),
                                   att_dst1[perm].reshape(1, f))
    msg1, den1 = run_sc(hpb1, as1, ad1, m1)
    h1, hpb2, as2, ad2, m2 = mid(h0, msg1, den1.reshape(_NC, n, 1),
                                 b1.reshape(1, f), W2[:, perm],
                                 att_src2[perm].reshape(1, f),
                                 att_dst2[perm].reshape(1, f))
    msg2, den2 = run_sc(hpb2, as2, ad2, m2)
    return epilogue(h1, msg2, den2.reshape(_NC, n, 1), b2.reshape(1, f))


# speedup vs baseline: 1.0079x; 1.0079x over previous
import dataclasses
import functools

import jax
import jax.numpy as jnp
from jax import lax
from jax.experimental import pallas as pl
from jax.experimental.pallas import tpu as pltpu
from jax.experimental.pallas import tpu_sc as plsc

_NC = 2
_NS = 16
_NW = _NC * _NS
_LANE = 128


def _round_up(v, m):
    return (v + m - 1) // m * m


def _layer_tail(n, h, wp_ref, as_ref, ad_ref, h_ref, hpb_ref, asp_ref,
                adp_ref, m_ref):
    h_ref[...] = h
    hpp = jnp.dot(h, wp_ref[...], preferred_element_type=jnp.float32)
    a_s = jnp.sum(hpp * as_ref[...], axis=1, keepdims=True)
    a_d = jnp.sum(hpp * ad_ref[...], axis=1, keepdims=True)
    asp_ref[0:n, :] = a_s
    asp_ref[n:, :] = jnp.full((asp_ref.shape[0] - n, 1), -1e30, jnp.float32)
    adp_ref[...] = a_d
    hpb_ref[0:n, :] = hpp.astype(jnp.bfloat16)
    hpb_ref[n:, :] = jnp.zeros((hpb_ref.shape[0] - n, hpp.shape[1]),
                               jnp.bfloat16)
    mm = jnp.max(a_s) + jnp.max(a_d)
    m_ref[...] = jnp.broadcast_to(jnp.where(mm >= 0, mm, 0.2 * mm), (1, 16))


def _first_body(n, x_ref, w0_ref, b0_ref, wp_ref, as_ref, ad_ref,
                h_ref, hpb_ref, asp_ref, adp_ref, m_ref):
    acc = jnp.dot(x_ref[...], w0_ref[...],
                  preferred_element_type=jnp.float32)
    h = jnp.maximum(acc + b0_ref[...], 0.0)
    _layer_tail(n, h, wp_ref, as_ref, ad_ref, h_ref, hpb_ref, asp_ref,
                adp_ref, m_ref)


def _mid_body(n, h0_ref, msg_ref, den_ref, b_ref, wp_ref, as_ref, ad_ref,
              h_ref, hpb_ref, asp_ref, adp_ref, m_ref):
    sm = msg_ref[0] + msg_ref[1]
    d = den_ref[0] + den_ref[1]
    h = h0_ref[...] + jnp.maximum(sm / (d + 1e-16) + b_ref[...], 0.0)
    _layer_tail(n, h, wp_ref, as_ref, ad_ref, h_ref, hpb_ref, asp_ref,
                adp_ref, m_ref)


def _epilogue_body(h_ref, msg_ref, den_ref, b_ref, o_ref):
    sm = msg_ref[0] + msg_ref[1]
    d = den_ref[0] + den_ref[1]
    o_ref[...] = h_ref[...] + jnp.maximum(sm / (d + 1e-16) + b_ref[...], 0.0)


def _make_sc_edge_pass(n, f, rw):
    zc = (n // _NS) & ~7
    zt = n - _NS * zc
    on = n // _NS
    mesh = plsc.VectorSubcoreMesh(core_axis_name="c", subcore_axis_name="s")
    cp = pltpu.CompilerParams()
    if "needs_layout_passes" in pltpu.CompilerParams.__dataclass_fields__:
        cp = dataclasses.replace(cp, needs_layout_passes=False)
    if "use_tc_tiling_on_sc" in pltpu.CompilerParams.__dataclass_fields__:
        cp = dataclasses.replace(cp, use_tc_tiling_on_sc=False)

    @functools.partial(
        pl.kernel,
        mesh=mesh,
        compiler_params=cp,
        out_type=[
            jax.ShapeDtypeStruct((_NC, n, f), jnp.float32),
            jax.ShapeDtypeStruct((_NC, n), jnp.float32),
        ],
        scratch_types=[
            pltpu.VMEM((n + 16,), jnp.float32),
            pltpu.VMEM((n,), jnp.float32),
            pltpu.VMEM((16,), jnp.float32),
            pltpu.VMEM((rw, _LANE), jnp.int32),
            pltpu.VMEM((rw, _LANE), jnp.int32),
            pltpu.VMEM((4, _LANE, f), jnp.bfloat16),
            pltpu.VMEM((4, _LANE, f), jnp.float32),
            pltpu.VMEM((4, _LANE), jnp.float32),
            pltpu.VMEM_SHARED((n, f), jnp.float32),
            pltpu.VMEM_SHARED((n,), jnp.float32),
            pltpu.SemaphoreType.DMA((4,)),
            pltpu.SemaphoreType.DMA((4,)),
            pltpu.SemaphoreType.DMA((4,)),
        ],
    )
    def sc_edge_pass(hp_hbm, asrc_hbm, adst_hbm, m_hbm, src_hbm, dst_hbm,
                     z2_hbm, z1_hbm, msg_out, den_out, asrc_v, adst_v, m_v,
                     src_v, dst_v, rows_v, scat_v, ex_v, msg_acc, den_acc,
                     gsem, msem, dsem):
        c = lax.axis_index("c")
        s = lax.axis_index("s")
        w = s * _NC + c
        stage = [
            pltpu.async_copy(z2_hbm, msg_acc.at[pl.ds(s * zc, zc)],
                             msem.at[0]),
            pltpu.async_copy(z1_hbm, den_acc.at[pl.ds(s * zc, zc)],
                             msem.at[1]),
            pltpu.async_copy(asrc_hbm, asrc_v, msem.at[2]),
            pltpu.async_copy(adst_hbm, adst_v, msem.at[3]),
            pltpu.async_copy(m_hbm, m_v, dsem.at[0]),
            pltpu.async_copy(src_hbm.at[pl.ds(w * rw, rw)], src_v,
                             dsem.at[1]),
            pltpu.async_copy(dst_hbm.at[pl.ds(w * rw, rw)], dst_v,
                             dsem.at[2]),
        ]
        if zt:
            @pl.when(s == 0)
            def _zero_tail():
                pltpu.async_copy(z2_hbm.at[pl.ds(0, zt)],
                                 msg_acc.at[pl.ds(_NS * zc, zt)],
                                 dsem.at[3]).wait()
                pltpu.async_copy(z1_hbm.at[pl.ds(0, zt)],
                                 den_acc.at[pl.ds(_NS * zc, zt)],
                                 gsem.at[0]).wait()
        for cp in stage:
            cp.wait()
        plsc.subcore_barrier()
        mvec = m_v[...]

        def _drain_scatters(b, r):
            pltpu.make_async_copy(scat_v.at[b], msg_acc.at[dst_v.at[r]],
                                  msem.at[b]).wait()
            pltpu.make_async_copy(ex_v.at[b], den_acc.at[dst_v.at[r]],
                                  dsem.at[b]).wait()

        def _do_row(r, b):
            nb = (b + 3) % 4

            @pl.when(r + 3 < rw)
            def _prefetch():
                pltpu.async_copy(hp_hbm.at[src_v.at[r + 3]],
                                 rows_v.at[nb], gsem.at[nb])

            pltpu.make_async_copy(hp_hbm.at[src_v.at[r]], rows_v.at[b],
                                  gsem.at[b]).wait()

            @pl.when(r >= 4)
            def _drain():
                _drain_scatters(b, r)

            exb = ex_v.at[b]
            for k in range(_LANE // 16):
                sidx = src_v[r, pl.ds(k * 16, 16)]
                didx = dst_v[r, pl.ds(k * 16, 16)]
                a = (plsc.load_gather(asrc_v, [sidx])
                     + plsc.load_gather(adst_v, [didx]))
                a = jnp.where(a >= 0, a, a * 0.2)
                exb[pl.ds(k * 16, 16)] = jnp.exp(a - mvec)

            rv = rows_v.at[b]
            sv = scat_v.at[b]
            himask = jnp.full((16,), 0xFFFF0000, jnp.uint32)
            for i in range(_LANE):
                eb = plsc.load_gather(exb, [jnp.full((16,), i, jnp.int32)])
                for h in range(f // 32):
                    u = plsc.bitcast(rv[i, pl.ds(h * 32, 32)], jnp.uint32)
                    lo = plsc.bitcast(u << 16, jnp.float32)
                    hi = plsc.bitcast(u & himask, jnp.float32)
                    sv[i, pl.ds(h * 32, 16)] = lo * eb
                    sv[i, pl.ds(h * 32 + 16, 16)] = hi * eb

            pltpu.async_copy(exb, den_acc.at[dst_v.at[r]], dsem.at[b],
                             add=True)
            pltpu.async_copy(sv, msg_acc.at[dst_v.at[r]], msem.at[b],
                             add=True)

        for pb in range(3):
            pltpu.async_copy(hp_hbm.at[src_v.at[pb]], rows_v.at[pb],
                             gsem.at[pb])

        @pl.loop(0, rw // 4)
        def _edges(q):
            _do_row(4 * q, 0)
            _do_row(4 * q + 1, 1)
            _do_row(4 * q + 2, 2)
            _do_row(4 * q + 3, 3)

        for b in range(4):
            _drain_scatters(b, 0)
        plsc.subcore_barrier()
        sl = pl.ds(s * on, on)
        pltpu.sync_copy(msg_acc.at[sl], msg_out.at[c].at[sl])

        @pl.when(s == 0)
        def _den_out():
            pltpu.sync_copy(den_acc.at[pl.ds(0, n)], den_out.at[c])

    return sc_edge_pass


def kernel(x, edge_index, edge_weight, edge_attr, W0, b0, W1, att_src1,
           att_dst1, b1, W2, att_src2, att_dst2, b2):
    n, d = x.shape
    f = W0.shape[1]
    e = edge_index.shape[1]

    rows = _round_up(pl.cdiv(e, _LANE), _NW * 8)
    rw = rows // _NW
    ep = rows * _LANE

    src = edge_index[0]
    dst = edge_index[1]
    pad = ep - e
    src2d = jnp.concatenate([src, jnp.full((pad,), n, jnp.int32)]).reshape(
        rows, _LANE)
    dst2d = jnp.concatenate([dst, jnp.zeros((pad,), jnp.int32)]).reshape(
        rows, _LANE)
    z2 = jnp.zeros(((n // _NS) & ~7, f), jnp.float32)
    z1 = jnp.zeros(((n // _NS) & ~7,), jnp.float32)
    perm = [0] * f
    for h in range(f // 32):
        for k in range(16):
            perm[32 * h + 2 * k] = 32 * h + k
            perm[32 * h + 2 * k + 1] = 32 * h + 16 + k
    perm = jnp.asarray(perm, jnp.int32)

    sc_edge_pass = _make_sc_edge_pass(n, f, rw)

    pre_out = [
        jax.ShapeDtypeStruct((n, f), jnp.float32),
        jax.ShapeDtypeStruct((n + 16, f), jnp.bfloat16),
        jax.ShapeDtypeStruct((n + 16, 1), jnp.float32),
        jax.ShapeDtypeStruct((n, 1), jnp.float32),
        jax.ShapeDtypeStruct((1, 16), jnp.float32),
    ]
    first = pl.pallas_call(functools.partial(_first_body, n),
                           out_shape=pre_out)
    mid = pl.pallas_call(functools.partial(_mid_body, n), out_shape=pre_out)
    epilogue = pl.pallas_call(
        _epilogue_body,
        out_shape=jax.ShapeDtypeStruct((n, f), jnp.float32),
    )

    def run_sc(hpb, a_s, a_d, m):
        return sc_edge_pass(hpb, a_s.reshape(n + 16), a_d.reshape(n),
                            m.reshape(16), src2d, dst2d, z2, z1)

    h0, hpb1, as1, ad1, m1 = first(x, W0, b0.reshape(1, f), W1[:, perm],
                                   att_src1[perm].reshape(1, f),
                                   att_dst1[perm].reshape(1, f))
    msg1, den1 = run_sc(hpb1, as1, ad1, m1)
    h1, hpb2, as2, ad2, m2 = mid(h0, msg1, den1.reshape(_NC, n, 1),
                                 b1.reshape(1, f), W2[:, perm],
                                 att_src2[perm].reshape(1, f),
                                 att_dst2[perm].reshape(1, f))
    msg2, den2 = run_sc(hpb2, as2, ad2, m2)
    return epilogue(h1, msg2, den2.reshape(_NC, n, 1), b2.reshape(1, f))
